# baseline (speedup 1.0000x reference)
import functools

import numpy as np
import jax
import jax.numpy as jnp
from jax import lax
from jax.experimental import pallas as pl
from jax.experimental.pallas import tpu as pltpu
from jax.experimental.pallas import tpu_sc as plsc

N = 16777216
NC = 2
NS = 16
NW = NC * NS
L = 16
PER_W = N // NW
CHUNK = 65536
NCHUNK = PER_W // CHUNK
NVEC = CHUNK // L

LOWER = 0.0
UPPER = 0.8
LOG_PDF = float(-np.log(np.float32(UPPER) - np.float32(LOWER), dtype=np.float32))

@functools.cache
def _build_sc_kernel():
    mesh = plsc.VectorSubcoreMesh(core_axis_name="c", subcore_axis_name="s")

    @functools.partial(
        pl.kernel,
        mesh=mesh,
        out_type=jax.ShapeDtypeStruct((N,), jnp.float32),
        scratch_types=[
            pltpu.VMEM((CHUNK,), jnp.float32),
        ],
    )
    def _uniform_ll_sc(x_hbm, out_hbm, buf):
        wid = lax.axis_index("s") * NC + lax.axis_index("c")
        base = wid * PER_W

        def chunk_body(g, _):
            off = base + g * CHUNK
            pltpu.sync_copy(x_hbm.at[pl.ds(off, CHUNK)], buf)

            def vec_body(i, _):
                v = buf[pl.ds(i * L, L)]
                cond = jnp.logical_and(
                    v >= jnp.float32(LOWER), v < jnp.float32(UPPER)
                )
                buf[pl.ds(i * L, L)] = jnp.where(
                    cond, jnp.float32(LOG_PDF), jnp.float32(-jnp.inf)
                )
                return 0

            lax.fori_loop(0, NVEC, vec_body, 0)
            pltpu.sync_copy(buf, out_hbm.at[pl.ds(off, CHUNK)])
            return 0

        lax.fori_loop(0, NCHUNK, chunk_body, 0)

    return _uniform_ll_sc


def kernel(x):
    return _build_sc_kernel()(x.reshape(N))


# speedup vs baseline: 3.3366x; 3.3366x over previous
import functools

import numpy as np
import jax
import jax.numpy as jnp
from jax import lax
from jax.experimental import pallas as pl
from jax.experimental.pallas import tpu as pltpu
from jax.experimental.pallas import tpu_sc as plsc

N = 16777216
NC = 2
NS = 16
NW = NC * NS
L = 16
PER_W = N // NW
CHUNK = 16384
NCHUNK = PER_W // CHUNK

LOWER = 0.0
UPPER = 0.8
LOG_PDF = float(-np.log(np.float32(UPPER) - np.float32(LOWER), dtype=np.float32))


@functools.cache
def _build_sc_kernel():
    mesh = plsc.VectorSubcoreMesh(core_axis_name="c", subcore_axis_name="s")

    @functools.partial(
        pl.kernel,
        mesh=mesh,
        out_type=jax.ShapeDtypeStruct((N,), jnp.float32),
        scratch_types=[
            pltpu.VMEM((CHUNK,), jnp.float32),
            pltpu.VMEM((CHUNK,), jnp.float32),
            pltpu.VMEM((CHUNK,), jnp.float32),
            pltpu.VMEM((CHUNK,), jnp.float32),
            pltpu.SemaphoreType.DMA,
            pltpu.SemaphoreType.DMA,
            pltpu.SemaphoreType.DMA,
            pltpu.SemaphoreType.DMA,
        ],
    )
    def _uniform_ll_sc(x_hbm, out_hbm, in0, in1, out0, out1, si0, si1, so0, so1):
        wid = lax.axis_index("s") * NC + lax.axis_index("c")
        base = wid * PER_W
        ins, outs = (in0, in1), (out0, out1)
        sin, sout = (si0, si1), (so0, so1)

        def make_in(g):
            off = base + g * CHUNK
            return pltpu.make_async_copy(
                x_hbm.at[pl.ds(off, CHUNK)], ins[g % 2], sin[g % 2]
            )

        def make_out(g):
            off = base + g * CHUNK
            return pltpu.make_async_copy(
                outs[g % 2], out_hbm.at[pl.ds(off, CHUNK)], sout[g % 2]
            )

        def compute(g):
            inb, outb = ins[g % 2], outs[g % 2]

            @plsc.parallel_loop(0, CHUNK, step=L, unroll=8)
            def _(i):
                v = inb[pl.ds(i, L)]
                cond = jnp.logical_and(
                    v >= jnp.float32(LOWER), v < jnp.float32(UPPER)
                )
                outb[pl.ds(i, L)] = jnp.where(
                    cond, jnp.float32(LOG_PDF), jnp.float32(-jnp.inf)
                )

        in_copies = [make_in(g) for g in range(NCHUNK)]
        out_copies = [make_out(g) for g in range(NCHUNK)]

        in_copies[0].start()
        for g in range(NCHUNK):
            if g + 1 < NCHUNK:
                in_copies[g + 1].start()
            in_copies[g].wait()
            if g >= 2:
                out_copies[g - 2].wait()
            compute(g)
            out_copies[g].start()
        out_copies[NCHUNK - 2].wait()
        out_copies[NCHUNK - 1].wait()

    return _uniform_ll_sc


def kernel(x):
    return _build_sc_kernel()(x.reshape(N))
